# baseline (speedup 1.0000x reference)
import functools

import jax
import jax.numpy as jnp
from jax import lax
from jax.experimental import pallas as pl
from jax.experimental.pallas import tpu as pltpu
from jax.experimental.pallas import tpu_sc as plsc

_NC = 2
_NS = 16
_NW = _NC * _NS
_L = 16
_CHUNK = 128
_TOTAL = 8 * 16384
_NPW = _TOTAL // _NW
_NCH = _NPW // _CHUNK
_NBUF = 4
_BLK = 128
_WPB = _NW // 8


def _rsqrt(x):
    xi = plsc.bitcast(x, jnp.int32)
    yi = jnp.int32(0x5F3759DF) - (xi >> 1)
    y = plsc.bitcast(yi, jnp.float32)
    for _ in range(3):
        y = y * (1.5 - 0.5 * x * y * y)
    return y


def _make_kernel():
    mesh = plsc.VectorSubcoreMesh(core_axis_name="c", subcore_axis_name="s")

    @functools.partial(
        pl.kernel,
        mesh=mesh,
        compiler_params=pltpu.CompilerParams(needs_layout_passes=False),
        out_type=jax.ShapeDtypeStruct((_NW, _L), jnp.float32),
        scratch_types=[
            pltpu.VMEM((_NCH, _CHUNK), jnp.int32),
            pltpu.VMEM((_NCH, _CHUNK), jnp.int32),
            pltpu.VMEM((_CHUNK, _BLK), jnp.float32),
            pltpu.VMEM((_CHUNK, _BLK), jnp.float32),
            pltpu.VMEM((_CHUNK, _BLK), jnp.float32),
            pltpu.VMEM((_CHUNK, _BLK), jnp.float32),
            pltpu.VMEM((_L,), jnp.float32),
            pltpu.SemaphoreType.DMA,
            pltpu.SemaphoreType.DMA,
            pltpu.SemaphoreType.DMA,
            pltpu.SemaphoreType.DMA,
            pltpu.SemaphoreType.DMA,
        ],
    )
    def k(table_hbm, idx_hbm, out_hbm, idx_v, blk_v, buf0, buf1, buf2,
          buf3, acc_v, sem_i, sem0, sem1, sem2, sem3):
        bufs = (buf0, buf1, buf2, buf3)
        sems = (sem0, sem1, sem2, sem3)
        wid = lax.axis_index("s") * _NC + lax.axis_index("c")
        brow = wid // _WPB
        c0 = (wid % _WPB) * _NPW

        idx_copies = [
            pltpu.make_async_copy(
                idx_hbm.at[brow, pl.ds(c0 + j * _CHUNK, _CHUNK)],
                idx_v.at[j], sem_i)
            for j in range(_NCH)
        ]
        for cp in idx_copies:
            cp.start()
        for cp in idx_copies:
            cp.wait()

        for j in range(_NCH):
            for g in range(_CHUNK // _L):
                sl = pl.ds(g * _L, _L)
                blk_v[j, sl] = idx_v[j, sl] >> 3

        def fire(chunk, slot):
            return pltpu.async_copy(
                table_hbm.at[blk_v.at[chunk]], bufs[slot], sems[slot])

        for t in range(_NBUF):
            fire(t, t)

        iota = lax.iota(jnp.int32, _L)

        def compute_chunk(chunk, slot, acc):
            for g in range(_CHUNK // _L):
                sl = pl.ds(g * _L, _L)
                iv = idx_v[chunk, sl]
                sub = (iv & 7) * _L
                slots = iota + (g * _L)
                ssq = jnp.zeros((_L,), jnp.float32)
                for c in range(_L):
                    gth = plsc.load_gather(bufs[slot], [slots, sub + c])
                    ssq = ssq + gth * gth
                acc = acc + ssq * _rsqrt(ssq)
            return acc

        def group_body(g, acc):
            for b in range(_NBUF):
                chunk = g * _NBUF + b
                pltpu.make_async_copy(
                    table_hbm.at[blk_v.at[chunk]], bufs[b], sems[b]).wait()
                acc = compute_chunk(chunk, b, acc)

                @pl.when(chunk + _NBUF < _NCH)
                def _():
                    fire(chunk + _NBUF, b)
            return acc

        acc = lax.fori_loop(0, _NCH // _NBUF, group_body,
                            jnp.zeros((_L,), jnp.float32))
        acc_v[...] = acc
        pltpu.sync_copy(acc_v, out_hbm.at[wid])

    return k


_sc_kernel = _make_kernel()


def kernel(table, indices):
    table_blocks = table.reshape(table.shape[0] // 8, 8 * table.shape[1])
    partials = _sc_kernel(table_blocks, indices.astype(jnp.int32))
    return jnp.sum(partials) * (1.0 / _TOTAL)


# speedup vs baseline: 4.2065x; 4.2065x over previous
import functools

import jax
import jax.numpy as jnp
from jax import lax
from jax.experimental import pallas as pl
from jax.experimental.pallas import tpu as pltpu
from jax.experimental.pallas import tpu_sc as plsc

_NC = 2
_NS = 16
_NW = _NC * _NS
_L = 16
_V = 1000000
_TOTAL = 8 * 16384
_NPW = _TOTAL // _NW
_CHUNK = 128
_NCH = _NPW // _CHUNK
_SW = 2048
_NFULL = _V // _SW
_HALF = 512
_TAIL = 64
_HSTART = _NFULL * _SW
_TSTART = _HSTART + _HALF
_CPW = -(-_NFULL // _NW)
_WPB = _NW // 8


def _rsqrt(x):
    xi = plsc.bitcast(x, jnp.int32)
    yi = jnp.int32(0x5F3759DF) - (xi >> 1)
    y = plsc.bitcast(yi, jnp.float32)
    for _ in range(3):
        y = y * (1.5 - 0.5 * x * y * y)
    return y


_params = pltpu.CompilerParams(needs_layout_passes=False)


def _make_sweep():
    mesh = plsc.VectorSubcoreMesh(core_axis_name="c", subcore_axis_name="s")

    @functools.partial(
        pl.kernel,
        mesh=mesh,
        compiler_params=_params,
        out_type=jax.ShapeDtypeStruct((_V,), jnp.float32),
        scratch_types=[
            pltpu.VMEM((_L, _SW), jnp.float32),
            pltpu.VMEM((_L, _SW), jnp.float32),
            pltpu.VMEM((_SW,), jnp.float32),
            pltpu.VMEM((_SW,), jnp.float32),
            pltpu.VMEM((_L, _HALF), jnp.float32),
            pltpu.VMEM((_HALF,), jnp.float32),
            pltpu.VMEM((_L, _TAIL), jnp.float32),
            pltpu.VMEM((_TAIL,), jnp.float32),
            pltpu.SemaphoreType.DMA,
            pltpu.SemaphoreType.DMA,
            pltpu.SemaphoreType.DMA,
            pltpu.SemaphoreType.DMA,
        ],
    )
    def k(tabt_hbm, norms_hbm, tb0, tb1, nb0, nb1, tbh, nbh, tbt, nbt,
          si0, si1, so0, so1):
        tbufs, nbufs = (tb0, tb1), (nb0, nb1)
        sin, sout = (si0, si1), (so0, so1)
        wid = lax.axis_index("s") * _NC + lax.axis_index("c")

        def qstart(c):
            return pl.multiple_of((wid + _NW * c) * _SW, _SW)

        def valid(c):
            return wid + _NW * c < _NFULL

        def fire_in(c, b):
            return pltpu.async_copy(
                tabt_hbm.at[:, pl.ds(qstart(c), _SW)], tbufs[b], sin[b])

        fire_in(0, 0)
        fire_in(1, 1)

        def norm_groups(tb, nb, ngroups):
            def group(g, _):
                sl = pl.ds(g * _L, _L)
                ssq = jnp.zeros((_L,), jnp.float32)
                for p in range(_L):
                    v = tb[p, sl]
                    ssq = ssq + v * v
                nb[sl] = ssq * _rsqrt(ssq)
                return 0

            lax.fori_loop(0, ngroups, group, 0)

        def body(s, _):
            for b in range(2):
                c = s * 2 + b

                @pl.when(valid(c))
                def _():
                    @pl.when(c >= 2)
                    def _():
                        pltpu.make_async_copy(
                            nbufs[b],
                            norms_hbm.at[pl.ds(qstart(c - 2), _SW)],
                            sout[b]).wait()

                    pltpu.make_async_copy(
                        tabt_hbm.at[:, pl.ds(qstart(c), _SW)], tbufs[b],
                        sin[b]).wait()
                    norm_groups(tbufs[b], nbufs[b], _SW // _L)
                    pltpu.async_copy(
                        nbufs[b], norms_hbm.at[pl.ds(qstart(c), _SW)],
                        sout[b])

                    @pl.when(valid(c + 2))
                    def _():
                        fire_in(c + 2, b)
            return 0

        lax.fori_loop(0, _CPW // 2, body, 0)
        for b in range(2):
            c_last = _CPW - 2 + b

            @pl.when(valid(c_last))
            def _():
                pltpu.make_async_copy(
                    nbufs[b], norms_hbm.at[pl.ds(qstart(c_last), _SW)],
                    sout[b]).wait()

        @pl.when(wid == 8)
        def _():
            pltpu.sync_copy(tabt_hbm.at[:, pl.ds(_HSTART, _HALF)], tbh)
            norm_groups(tbh, nbh, _HALF // _L)
            pltpu.sync_copy(nbh, norms_hbm.at[pl.ds(_HSTART, _HALF)])

        @pl.when(wid == 9)
        def _():
            pltpu.sync_copy(tabt_hbm.at[:, pl.ds(_TSTART, _TAIL)], tbt)
            norm_groups(tbt, nbt, _TAIL // _L)
            pltpu.sync_copy(nbt, norms_hbm.at[pl.ds(_TSTART, _TAIL)])

    return k


def _make_gather():
    mesh = plsc.VectorSubcoreMesh(core_axis_name="c", subcore_axis_name="s")

    @functools.partial(
        pl.kernel,
        mesh=mesh,
        compiler_params=_params,
        out_type=jax.ShapeDtypeStruct((_NW, _L), jnp.float32),
        scratch_types=[
            pltpu.VMEM((_NCH, _CHUNK), jnp.int32),
            pltpu.VMEM((_NCH, _CHUNK), jnp.float32),
            pltpu.VMEM((_L,), jnp.float32),
            pltpu.SemaphoreType.DMA,
            pltpu.SemaphoreType.DMA,
        ],
    )
    def k(norms_hbm, idx_hbm, out_hbm, idx_v, nrm_v, acc_v, sem_i, sem_g):
        wid = lax.axis_index("s") * _NC + lax.axis_index("c")
        brow = wid // _WPB
        c0 = (wid % _WPB) * _NPW

        idx_copies = [
            pltpu.make_async_copy(
                idx_hbm.at[brow, pl.ds(c0 + j * _CHUNK, _CHUNK)],
                idx_v.at[j], sem_i)
            for j in range(_NCH)
        ]
        for cp in idx_copies:
            cp.start()
        for cp in idx_copies:
            cp.wait()

        gathers = [
            pltpu.make_async_copy(
                norms_hbm.at[idx_v.at[j]], nrm_v.at[j], sem_g)
            for j in range(_NCH)
        ]
        for cp in gathers:
            cp.start()
        for cp in gathers:
            cp.wait()

        def chunk_sum(j, acc):
            for g in range(_CHUNK // _L):
                acc = acc + nrm_v[j, pl.ds(g * _L, _L)]
            return acc

        acc = lax.fori_loop(0, _NCH, chunk_sum,
                            jnp.zeros((_L,), jnp.float32))
        acc_v[...] = acc
        pltpu.sync_copy(acc_v, out_hbm.at[wid])

    return k


_sweep_kernel = _make_sweep()
_gather_kernel = _make_gather()


def kernel(table, indices):
    norms = _sweep_kernel(table.T)
    partials = _gather_kernel(norms, indices.astype(jnp.int32))
    return jnp.sum(partials) * (1.0 / _TOTAL)
